# baseline (speedup 1.0000x reference)
import functools

import jax
import jax.numpy as jnp
from jax import lax
from jax.experimental import pallas as pl
from jax.experimental.pallas import tpu as pltpu
from jax.experimental.pallas import tpu_sc as plsc

_VOCAB = 100000
_C = 64
_B = 32
_T = 8
_NTOK = _B * _T

_NC = 2
_NS = 16
_NW = _NC * _NS
_RPW = _NTOK // _NW


def _sc_gather_body(pidx_hbm, pairs_hbm, xp_hbm, pidx_v, rows_v, sem):
    wid = lax.axis_index("s") * _NC + lax.axis_index("c")
    base = wid * _RPW
    pltpu.sync_copy(pidx_hbm.at[pl.ds(base, _RPW)], pidx_v)
    pltpu.async_copy(pairs_hbm.at[pidx_v], rows_v, sem).wait()
    pltpu.sync_copy(rows_v, xp_hbm.at[pl.ds(base, _RPW)])


_sc_gather = functools.partial(
    pl.kernel,
    mesh=plsc.VectorSubcoreMesh(core_axis_name="c", subcore_axis_name="s"),
    out_type=jax.ShapeDtypeStruct((_NTOK, 2 * _C), jnp.float32),
    scratch_types=[
        pltpu.VMEM((_RPW,), jnp.int32),
        pltpu.VMEM((_RPW, 2 * _C), jnp.float32),
        pltpu.SemaphoreType.DMA,
    ],
)(_sc_gather_body)


_PACK_BLK = 2000


def _pack_body(t_ref, p_ref):
    t = t_ref[...].reshape(_PACK_BLK, 2, _C)
    p_ref[...] = jnp.concatenate([t[:, 0, :], t[:, 1, :]], axis=1)


def _pack(tok_table):
    grid = (_VOCAB // 2 // _PACK_BLK,)
    return pl.pallas_call(
        _pack_body,
        grid=grid,
        in_specs=[pl.BlockSpec((2 * _PACK_BLK, _C), lambda i: (i, 0))],
        out_specs=pl.BlockSpec((_PACK_BLK, 2 * _C), lambda i: (i, 0)),
        out_shape=jax.ShapeDtypeStruct((_VOCAB // 2, 2 * _C), jnp.float32),
        compiler_params=pltpu.CompilerParams(
            dimension_semantics=("parallel",),
        ),
    )(tok_table)


_N_TILE = 16384


def _mm_body(xp_ref, oha_ref, posb_ref, w_ref, b_ref, o_ref, x_scratch):
    @pl.when(pl.program_id(0) == 0)
    def _():
        oha = oha_ref[...]
        x_scratch[...] = (
            xp_ref[:, : _C] * oha
            + xp_ref[:, _C :] * (1.0 - oha)
            + posb_ref[...]
        )

    o_ref[...] = (
        jnp.dot(x_scratch[...], w_ref[...], preferred_element_type=jnp.float32)
        + b_ref[...]
    )


def _head(xp, oha, posb, W, b2):
    grid = (pl.cdiv(_VOCAB, _N_TILE),)
    return pl.pallas_call(
        _mm_body,
        grid=grid,
        in_specs=[
            pl.BlockSpec((_NTOK, 2 * _C), lambda i: (0, 0)),
            pl.BlockSpec((_NTOK, 1), lambda i: (0, 0)),
            pl.BlockSpec((_NTOK, _C), lambda i: (0, 0)),
            pl.BlockSpec((_C, _N_TILE), lambda i: (0, i)),
            pl.BlockSpec((1, _N_TILE), lambda i: (0, i)),
        ],
        out_specs=pl.BlockSpec((_NTOK, _N_TILE), lambda i: (0, i)),
        out_shape=jax.ShapeDtypeStruct((_NTOK, _VOCAB), jnp.float32),
        scratch_shapes=[pltpu.VMEM((_NTOK, _C), jnp.float32)],
        compiler_params=pltpu.CompilerParams(
            dimension_semantics=("arbitrary",),
        ),
    )(xp, oha, posb, W, b2)


def kernel(idx, tok_table, pos_table, W, b):
    idx_flat = idx.reshape(-1).astype(jnp.int32)
    pairs = _pack(tok_table)
    xp = _sc_gather(idx_flat >> 1, pairs)
    oha = ((idx_flat & 1) == 0).astype(jnp.float32)[:, None]
    posb = jnp.tile(pos_table[:_T], (_B, 1))
    logits = _head(xp, oha, posb, W, b.reshape(1, -1))
    return logits.reshape(_B, _T, _VOCAB)


# speedup vs baseline: 1.1627x; 1.1627x over previous
import functools

import jax
import jax.numpy as jnp
from jax import lax
from jax.experimental import pallas as pl
from jax.experimental.pallas import tpu as pltpu
from jax.experimental.pallas import tpu_sc as plsc

_VOCAB = 100000
_C = 64
_B = 32
_T = 8
_NTOK = _B * _T

_NC = 2
_NS = 16
_NW = _NC * _NS
_RPW = _NTOK // _NW


def _sc_gather_body(pidx_hbm, pairs_hbm, xp_hbm, pidx_v, rows_v, sem):
    wid = lax.axis_index("s") * _NC + lax.axis_index("c")
    base = wid * _RPW
    pltpu.sync_copy(pidx_hbm.at[pl.ds(base, _RPW)], pidx_v)
    pltpu.async_copy(pairs_hbm.at[pidx_v], rows_v, sem).wait()
    pltpu.sync_copy(rows_v, xp_hbm.at[pl.ds(base, _RPW)])


_sc_gather = functools.partial(
    pl.kernel,
    mesh=plsc.VectorSubcoreMesh(core_axis_name="c", subcore_axis_name="s"),
    out_type=jax.ShapeDtypeStruct((_NTOK, 2 * _C), jnp.float32),
    scratch_types=[
        pltpu.VMEM((_RPW,), jnp.int32),
        pltpu.VMEM((_RPW, 2 * _C), jnp.float32),
        pltpu.SemaphoreType.DMA,
    ],
)(_sc_gather_body)


_N_TILE = 16384


def _mm_body(xp_ref, oha_ref, posb_ref, w_ref, b_ref, o_ref, x_scratch):
    @pl.when(pl.program_id(0) == 0)
    def _():
        oha = oha_ref[...]
        x_scratch[...] = (
            xp_ref[:, : _C] * oha
            + xp_ref[:, _C :] * (1.0 - oha)
            + posb_ref[...]
        )

    o_ref[...] = (
        jnp.dot(x_scratch[...], w_ref[...], preferred_element_type=jnp.float32)
        + b_ref[...]
    )


def _head(xp, oha, posb, W, b2):
    grid = (pl.cdiv(_VOCAB, _N_TILE),)
    return pl.pallas_call(
        _mm_body,
        grid=grid,
        in_specs=[
            pl.BlockSpec((_NTOK, 2 * _C), lambda i: (0, 0)),
            pl.BlockSpec((_NTOK, 1), lambda i: (0, 0)),
            pl.BlockSpec((_NTOK, _C), lambda i: (0, 0)),
            pl.BlockSpec((_C, _N_TILE), lambda i: (0, i)),
            pl.BlockSpec((1, _N_TILE), lambda i: (0, i)),
        ],
        out_specs=pl.BlockSpec((_NTOK, _N_TILE), lambda i: (0, i)),
        out_shape=jax.ShapeDtypeStruct((_NTOK, _VOCAB), jnp.float32),
        scratch_shapes=[pltpu.VMEM((_NTOK, _C), jnp.float32)],
        compiler_params=pltpu.CompilerParams(
            dimension_semantics=("arbitrary",),
        ),
    )(xp, oha, posb, W, b2)


def kernel(idx, tok_table, pos_table, W, b):
    idx_flat = idx.reshape(-1).astype(jnp.int32)
    pairs = tok_table.reshape(_VOCAB // 2, 2 * _C)
    xp = _sc_gather(idx_flat >> 1, pairs)
    oha = ((idx_flat & 1) == 0).astype(jnp.float32)[:, None]
    posb = jnp.tile(pos_table[:_T], (_B, 1))
    logits = _head(xp, oha, posb, W, b.reshape(1, -1))
    return logits.reshape(_B, _T, _VOCAB)
